# Initial kernel scaffold; baseline (speedup 1.0000x reference)
import jax
import jax.numpy as jnp
from jax.experimental import pallas as pl


def kernel(input_ids, token_type_ids, event_type_ids, word_table, pos_table, type_table, event_table, gamma, beta):
    raise NotImplementedError("write your pallas kernel here")


# speedup vs baseline: 3.9679x; 3.9679x over previous
import functools

import jax
import jax.numpy as jnp
from jax import lax
from jax.experimental import pallas as pl
from jax.experimental.pallas import tpu as pltpu
from jax.experimental.pallas import tpu_sc as plsc

B, S, H = 1024, 200, 128
V, P, T, E = 100000, 512, 2, 10
EPS = 1e-12
BS = B * S

NC, NS = 2, 16
NW = NC * NS
TOK_W = BS // NW
CH = 128
NIT = TOK_W // CH


def _sc_gather(word_table, ids_flat):
    mesh = plsc.VectorSubcoreMesh(core_axis_name="c", subcore_axis_name="s")

    @functools.partial(
        pl.kernel,
        mesh=mesh,
        out_type=jax.ShapeDtypeStruct((BS, H), jnp.float32),
        scratch_types=[
            pltpu.VMEM((CH,), jnp.int32),
            pltpu.VMEM((CH, H), jnp.float32),
            pltpu.SemaphoreType.DMA,
        ],
    )
    def k(table_hbm, ids_hbm, out_hbm, idx_v, rows_v, sem):
        wid = lax.axis_index("s") * NC + lax.axis_index("c")

        def body(j, carry):
            base = wid * TOK_W + j * CH
            pltpu.sync_copy(ids_hbm.at[pl.ds(base, CH)], idx_v)
            pltpu.async_copy(table_hbm.at[idx_v], rows_v, sem).wait()
            pltpu.sync_copy(rows_v, out_hbm.at[pl.ds(base, CH)])
            return carry

        lax.fori_loop(0, NIT, body, 0)

    return k(word_table, ids_flat)


BB = 8


def _tc_norm(word_emb, tt_ids, ev_ids, pos_s, type_table, event_table, gamma, beta):
    def body(we_ref, tt_ref, ev_ref, pos_ref, typ_ref, evt_ref, g_ref, b_ref, o_ref):
        shape3 = (BB, S, H)

        def bcast_rows(m2):
            return lax.broadcast_in_dim(m2, shape3, (0, 1))

        def bcast_lane(row):
            return lax.broadcast_in_dim(row, shape3, (2,))

        x = we_ref[...] + lax.broadcast_in_dim(pos_ref[...], shape3, (1, 2))
        tt = tt_ref[...]
        ev = ev_ref[...]
        for t in range(T):
            m = (tt == t).astype(jnp.float32)
            x = x + bcast_rows(m) * bcast_lane(typ_ref[t, :])
        for e in range(E):
            m = (ev == e).astype(jnp.float32)
            x = x + bcast_rows(m) * bcast_lane(evt_ref[e, :])
        mean = bcast_rows(jnp.mean(x, axis=-1))
        xc = x - mean
        var = bcast_rows(jnp.mean(xc * xc, axis=-1))
        o_ref[...] = (xc * lax.rsqrt(var + EPS) * bcast_lane(g_ref[...])
                      + bcast_lane(b_ref[...]))

    return pl.pallas_call(
        body,
        grid=(B // BB,),
        in_specs=[
            pl.BlockSpec((BB, S, H), lambda i: (i, 0, 0)),
            pl.BlockSpec((BB, S), lambda i: (i, 0)),
            pl.BlockSpec((BB, S), lambda i: (i, 0)),
            pl.BlockSpec((S, H), lambda i: (0, 0)),
            pl.BlockSpec((T, H), lambda i: (0, 0)),
            pl.BlockSpec((E, H), lambda i: (0, 0)),
            pl.BlockSpec((H,), lambda i: (0,)),
            pl.BlockSpec((H,), lambda i: (0,)),
        ],
        out_specs=pl.BlockSpec((BB, S, H), lambda i: (i, 0, 0)),
        out_shape=jax.ShapeDtypeStruct((B, S, H), jnp.float32),
    )(word_emb, tt_ids, ev_ids, pos_s, type_table, event_table, gamma, beta)


def kernel(input_ids, token_type_ids, event_type_ids, word_table, pos_table,
           type_table, event_table, gamma, beta):
    ids_flat = input_ids.reshape(BS).astype(jnp.int32)
    gathered = _sc_gather(word_table, ids_flat).reshape(B, S, H)
    return _tc_norm(gathered, token_type_ids, event_type_ids, pos_table[:S],
                    type_table, event_table, gamma, beta)


# speedup vs baseline: 5.8865x; 1.4835x over previous
import functools

import jax
import jax.numpy as jnp
from jax import lax
from jax.experimental import pallas as pl
from jax.experimental.pallas import tpu as pltpu
from jax.experimental.pallas import tpu_sc as plsc

B, S, H = 1024, 200, 128
V, P, T, E = 100000, 512, 2, 10
EPS = 1e-12
BS = B * S

NC, NS = 2, 16
NW = NC * NS
TOK_W = BS // NW
CH = 128
NIT = TOK_W // CH


def _sc_gather(word_table, ids_flat):
    mesh = plsc.VectorSubcoreMesh(core_axis_name="c", subcore_axis_name="s")

    @functools.partial(
        pl.kernel,
        mesh=mesh,
        out_type=jax.ShapeDtypeStruct((BS, H), jnp.float32),
        scratch_types=[
            pltpu.VMEM((CH,), jnp.int32),
            pltpu.VMEM((CH, H), jnp.float32),
            pltpu.SemaphoreType.DMA,
        ],
    )
    def k(table_hbm, ids_hbm, out_hbm, idx_v, rows_v, sem):
        wid = lax.axis_index("s") * NC + lax.axis_index("c")

        def body(j, carry):
            base = wid * TOK_W + j * CH
            pltpu.sync_copy(ids_hbm.at[pl.ds(base, CH)], idx_v)
            pltpu.async_copy(table_hbm.at[idx_v], rows_v, sem).wait()
            pltpu.sync_copy(rows_v, out_hbm.at[pl.ds(base, CH)])
            return carry

        lax.fori_loop(0, NIT, body, 0)

    return k(word_table, ids_flat)


BB = 8


def _tc_norm(word_emb, tt_ids, ev_ids, pos_s, type_table, event_table, gamma, beta):
    def body(we_ref, tt_ref, ev_ref, pos_ref, typ_ref, evt_ref, g_ref, b_ref, o_ref):
        shape3 = (BB, S, H)

        def bcast_rows(m2):
            return lax.broadcast_in_dim(m2, shape3, (0, 1))

        def bcast_lane(row):
            return lax.broadcast_in_dim(row, shape3, (2,))

        x = we_ref[...] + lax.broadcast_in_dim(pos_ref[...], shape3, (1, 2))

        def onehot_matmul(ids2, n, tab):
            idb = lax.broadcast_in_dim(ids2, (BB, S, n), (0, 1))
            iot = lax.broadcasted_iota(jnp.int32, (BB, S, n), 2)
            oh = (idb == iot).astype(jnp.float32).reshape(BB * S, n)
            r = jax.lax.dot_general(oh, tab, (((1,), (0,)), ((), ())),
                                    precision=lax.Precision.HIGHEST,
                                    preferred_element_type=jnp.float32)
            return r.reshape(BB, S, H)

        x = x + onehot_matmul(tt_ref[...], T, typ_ref[...])
        x = x + onehot_matmul(ev_ref[...], E, evt_ref[...])
        mean = bcast_rows(jnp.mean(x, axis=-1))
        xc = x - mean
        var = bcast_rows(jnp.mean(xc * xc, axis=-1))
        o_ref[...] = (xc * lax.rsqrt(var + EPS) * bcast_lane(g_ref[...])
                      + bcast_lane(b_ref[...]))

    return pl.pallas_call(
        body,
        grid=(B // BB,),
        in_specs=[
            pl.BlockSpec((BB, S, H), lambda i: (i, 0, 0)),
            pl.BlockSpec((BB, S), lambda i: (i, 0)),
            pl.BlockSpec((BB, S), lambda i: (i, 0)),
            pl.BlockSpec((S, H), lambda i: (0, 0)),
            pl.BlockSpec((T, H), lambda i: (0, 0)),
            pl.BlockSpec((E, H), lambda i: (0, 0)),
            pl.BlockSpec((H,), lambda i: (0,)),
            pl.BlockSpec((H,), lambda i: (0,)),
        ],
        out_specs=pl.BlockSpec((BB, S, H), lambda i: (i, 0, 0)),
        out_shape=jax.ShapeDtypeStruct((B, S, H), jnp.float32),
    )(word_emb, tt_ids, ev_ids, pos_s, type_table, event_table, gamma, beta)


def kernel(input_ids, token_type_ids, event_type_ids, word_table, pos_table,
           type_table, event_table, gamma, beta):
    ids_flat = input_ids.reshape(BS).astype(jnp.int32)
    gathered = _sc_gather(word_table, ids_flat).reshape(B, S, H)
    return _tc_norm(gathered, token_type_ids, event_type_ids, pos_table[:S],
                    type_table, event_table, gamma, beta)


# speedup vs baseline: 7.1238x; 1.2102x over previous
import functools

import jax
import jax.numpy as jnp
from jax import lax
from jax.experimental import pallas as pl
from jax.experimental.pallas import tpu as pltpu
from jax.experimental.pallas import tpu_sc as plsc

B, S, H = 1024, 200, 128
V, P, T, E = 100000, 512, 2, 10
EPS = 1e-12
BS = B * S

NC, NS = 2, 16
NW = NC * NS
TOK_W = BS // NW
CH = 128
NIT = TOK_W // CH


def _sc_gather(word_table, ids_flat):
    mesh = plsc.VectorSubcoreMesh(core_axis_name="c", subcore_axis_name="s")

    @functools.partial(
        pl.kernel,
        mesh=mesh,
        out_type=jax.ShapeDtypeStruct((BS, H), jnp.float32),
        scratch_types=[
            pltpu.VMEM((CH,), jnp.int32),
            pltpu.VMEM((CH, H), jnp.float32),
            pltpu.SemaphoreType.DMA,
        ],
    )
    def k(table_hbm, ids_hbm, out_hbm, idx_v, rows_v, sem):
        wid = lax.axis_index("s") * NC + lax.axis_index("c")

        def body(j, carry):
            base = wid * TOK_W + j * CH
            pltpu.sync_copy(ids_hbm.at[pl.ds(base, CH)], idx_v)
            pltpu.async_copy(table_hbm.at[idx_v], rows_v, sem).wait()
            pltpu.sync_copy(rows_v, out_hbm.at[pl.ds(base, CH)])
            return carry

        lax.fori_loop(0, NIT, body, 0)

    return k(word_table, ids_flat)


BB = 8


def _tc_norm(word_emb, tt_ids, ev_ids, pos_s, type_table, evt_hi, evt_lo,
             gamma, beta):
    def body(we_ref, tt_ref, ev_ref, pos_ref, typ_ref, evh_ref, evl_ref,
             g_ref, b_ref, o_ref):
        shape3 = (BB, S, H)

        def bcast_rows(m2):
            return lax.broadcast_in_dim(m2, shape3, (0, 1))

        def bcast_lane(row):
            return lax.broadcast_in_dim(row, shape3, (2,))

        p2 = pos_ref[...] + lax.broadcast_in_dim(typ_ref[0, :], (S, H), (1,))
        tdelta = typ_ref[1, :] - typ_ref[0, :]
        ttf = tt_ref[...].astype(jnp.float32)
        x = (we_ref[...] + lax.broadcast_in_dim(p2, shape3, (1, 2))
             + bcast_rows(ttf) * bcast_lane(tdelta))

        idb = lax.broadcast_in_dim(ev_ref[...], (BB, S, E), (0, 1))
        iot = lax.broadcasted_iota(jnp.int32, (BB, S, E), 2)
        oh = (idb == iot).astype(jnp.bfloat16).reshape(BB * S, E)
        dims = (((1,), (0,)), ((), ()))
        r = (jax.lax.dot_general(oh, evh_ref[...], dims,
                                 preferred_element_type=jnp.float32)
             + jax.lax.dot_general(oh, evl_ref[...], dims,
                                   preferred_element_type=jnp.float32))
        x = x + r.reshape(BB, S, H)

        mean = bcast_rows(jnp.mean(x, axis=-1))
        xc = x - mean
        var = bcast_rows(jnp.mean(xc * xc, axis=-1))
        o_ref[...] = (xc * lax.rsqrt(var + EPS) * bcast_lane(g_ref[...])
                      + bcast_lane(b_ref[...]))

    return pl.pallas_call(
        body,
        grid=(B // BB,),
        in_specs=[
            pl.BlockSpec((BB, S, H), lambda i: (i, 0, 0)),
            pl.BlockSpec((BB, S), lambda i: (i, 0)),
            pl.BlockSpec((BB, S), lambda i: (i, 0)),
            pl.BlockSpec((S, H), lambda i: (0, 0)),
            pl.BlockSpec((T, H), lambda i: (0, 0)),
            pl.BlockSpec((E, H), lambda i: (0, 0)),
            pl.BlockSpec((E, H), lambda i: (0, 0)),
            pl.BlockSpec((H,), lambda i: (0,)),
            pl.BlockSpec((H,), lambda i: (0,)),
        ],
        out_specs=pl.BlockSpec((BB, S, H), lambda i: (i, 0, 0)),
        out_shape=jax.ShapeDtypeStruct((B, S, H), jnp.float32),
    )(word_emb, tt_ids, ev_ids, pos_s, type_table, evt_hi, evt_lo,
      gamma, beta)


def kernel(input_ids, token_type_ids, event_type_ids, word_table, pos_table,
           type_table, event_table, gamma, beta):
    ids_flat = input_ids.reshape(BS).astype(jnp.int32)
    gathered = _sc_gather(word_table, ids_flat).reshape(B, S, H)
    evt_hi = event_table.astype(jnp.bfloat16)
    evt_lo = (event_table - evt_hi.astype(jnp.float32)).astype(jnp.bfloat16)
    return _tc_norm(gathered, token_type_ids, event_type_ids, pos_table[:S],
                    type_table, evt_hi, evt_lo, gamma, beta)


# speedup vs baseline: 8.0371x; 1.1282x over previous
import functools

import jax
import jax.numpy as jnp
from jax import lax
from jax.experimental import pallas as pl
from jax.experimental.pallas import tpu as pltpu
from jax.experimental.pallas import tpu_sc as plsc

B, S, H = 1024, 200, 128
V, P, T, E = 100000, 512, 2, 10
EPS = 1e-12
BS = B * S

NC, NS = 2, 16
NW = NC * NS
TOK_W = BS // NW
CH = 128
NIT = TOK_W // CH


def _sc_gather(word_table, ids_flat):
    mesh = plsc.VectorSubcoreMesh(core_axis_name="c", subcore_axis_name="s")

    @functools.partial(
        pl.kernel,
        mesh=mesh,
        out_type=jax.ShapeDtypeStruct((BS, H), jnp.float32),
        scratch_types=[
            pltpu.VMEM((CH,), jnp.int32),
            pltpu.VMEM((CH, H), jnp.float32),
            pltpu.SemaphoreType.DMA,
        ],
    )
    def k(table_hbm, ids_hbm, out_hbm, idx_v, rows_v, sem):
        wid = lax.axis_index("s") * NC + lax.axis_index("c")

        def body(j, carry):
            base = wid * TOK_W + j * CH
            pltpu.sync_copy(ids_hbm.at[pl.ds(base, CH)], idx_v)
            pltpu.async_copy(table_hbm.at[idx_v], rows_v, sem).wait()
            pltpu.sync_copy(rows_v, out_hbm.at[pl.ds(base, CH)])
            return carry

        lax.fori_loop(0, NIT, body, 0)

    return k(word_table, ids_flat)


BB = 16


def _tc_norm(word_emb, tt_ids, ev_ids, pos_s, type_table, evt_hi, evt_lo,
             gamma, beta):
    def body(we_ref, tt_ref, ev_ref, pos_ref, typ_ref, evh_ref, evl_ref,
             g_ref, b_ref, o_ref):
        shape3 = (BB, S, H)

        def bcast_rows(m2):
            return lax.broadcast_in_dim(m2, shape3, (0, 1))

        def bcast_lane(row):
            return lax.broadcast_in_dim(row, shape3, (2,))

        p2 = pos_ref[...] + lax.broadcast_in_dim(typ_ref[0, :], (S, H), (1,))
        tdelta = typ_ref[1, :] - typ_ref[0, :]
        ttf = tt_ref[...].astype(jnp.float32)
        x = (we_ref[...] + lax.broadcast_in_dim(p2, shape3, (1, 2))
             + bcast_rows(ttf) * bcast_lane(tdelta))

        idb = lax.broadcast_in_dim(ev_ref[...], (BB, S, E), (0, 1))
        iot = lax.broadcasted_iota(jnp.int32, (BB, S, E), 2)
        oh = (idb == iot).astype(jnp.bfloat16).reshape(BB * S, E)
        dims = (((1,), (0,)), ((), ()))
        r = (jax.lax.dot_general(oh, evh_ref[...], dims,
                                 preferred_element_type=jnp.float32)
             + jax.lax.dot_general(oh, evl_ref[...], dims,
                                   preferred_element_type=jnp.float32))
        x = x + r.reshape(BB, S, H)

        mean = bcast_rows(jnp.mean(x, axis=-1))
        xc = x - mean
        var = bcast_rows(jnp.mean(xc * xc, axis=-1))
        o_ref[...] = (xc * lax.rsqrt(var + EPS) * bcast_lane(g_ref[...])
                      + bcast_lane(b_ref[...]))

    return pl.pallas_call(
        body,
        grid=(B // BB,),
        in_specs=[
            pl.BlockSpec((BB, S, H), lambda i: (i, 0, 0)),
            pl.BlockSpec((BB, S), lambda i: (i, 0)),
            pl.BlockSpec((BB, S), lambda i: (i, 0)),
            pl.BlockSpec((S, H), lambda i: (0, 0)),
            pl.BlockSpec((T, H), lambda i: (0, 0)),
            pl.BlockSpec((E, H), lambda i: (0, 0)),
            pl.BlockSpec((E, H), lambda i: (0, 0)),
            pl.BlockSpec((H,), lambda i: (0,)),
            pl.BlockSpec((H,), lambda i: (0,)),
        ],
        out_specs=pl.BlockSpec((BB, S, H), lambda i: (i, 0, 0)),
        out_shape=jax.ShapeDtypeStruct((B, S, H), jnp.float32),
    )(word_emb, tt_ids, ev_ids, pos_s, type_table, evt_hi, evt_lo,
      gamma, beta)


def kernel(input_ids, token_type_ids, event_type_ids, word_table, pos_table,
           type_table, event_table, gamma, beta):
    ids_flat = input_ids.reshape(BS).astype(jnp.int32)
    gathered = _sc_gather(word_table, ids_flat).reshape(B, S, H)
    evt_hi = event_table.astype(jnp.bfloat16)
    evt_lo = (event_table - evt_hi.astype(jnp.float32)).astype(jnp.bfloat16)
    return _tc_norm(gathered, token_type_ids, event_type_ids, pos_table[:S],
                    type_table, evt_hi, evt_lo, gamma, beta)


# speedup vs baseline: 8.3227x; 1.0355x over previous
import functools

import jax
import jax.numpy as jnp
from jax import lax
from jax.experimental import pallas as pl
from jax.experimental.pallas import tpu as pltpu
from jax.experimental.pallas import tpu_sc as plsc

B, S, H = 1024, 200, 128
V, P, T, E = 100000, 512, 2, 10
EPS = 1e-12
BS = B * S

NC, NS = 2, 16
NW = NC * NS
TOK_W = BS // NW
CH = 128
NIT = TOK_W // CH


def _sc_gather(word_table, ids_flat):
    mesh = plsc.VectorSubcoreMesh(core_axis_name="c", subcore_axis_name="s")

    @functools.partial(
        pl.kernel,
        mesh=mesh,
        out_type=jax.ShapeDtypeStruct((BS, H), jnp.float32),
        scratch_types=[
            pltpu.VMEM((CH,), jnp.int32),
            pltpu.VMEM((CH, H), jnp.float32),
            pltpu.SemaphoreType.DMA,
        ],
    )
    def k(table_hbm, ids_hbm, out_hbm, idx_v, rows_v, sem):
        wid = lax.axis_index("s") * NC + lax.axis_index("c")

        def body(j, carry):
            base = wid * TOK_W + j * CH
            pltpu.sync_copy(ids_hbm.at[pl.ds(base, CH)], idx_v)
            pltpu.async_copy(table_hbm.at[idx_v], rows_v, sem).wait()
            pltpu.sync_copy(rows_v, out_hbm.at[pl.ds(base, CH)])
            return carry

        lax.fori_loop(0, NIT, body, 0)

    return k(word_table, ids_flat)


BB = 32


def _tc_norm(word_emb, tt_ids, ev_ids, pos_s, type_table, evt_hi, evt_lo,
             gamma, beta):
    def body(we_ref, tt_ref, ev_ref, pos_ref, typ_ref, evh_ref, evl_ref,
             g_ref, b_ref, o_ref):
        shape3 = (BB, S, H)

        def bcast_rows(m2):
            return lax.broadcast_in_dim(m2, shape3, (0, 1))

        def bcast_lane(row):
            return lax.broadcast_in_dim(row, shape3, (2,))

        p2 = pos_ref[...] + lax.broadcast_in_dim(typ_ref[0, :], (S, H), (1,))
        tdelta = typ_ref[1, :] - typ_ref[0, :]
        ttf = tt_ref[...].astype(jnp.float32)
        x = (we_ref[...] + lax.broadcast_in_dim(p2, shape3, (1, 2))
             + bcast_rows(ttf) * bcast_lane(tdelta))

        idb = lax.broadcast_in_dim(ev_ref[...], (BB, S, E), (0, 1))
        iot = lax.broadcasted_iota(jnp.int32, (BB, S, E), 2)
        oh = (idb == iot).astype(jnp.bfloat16).reshape(BB * S, E)
        dims = (((1,), (0,)), ((), ()))
        r = (jax.lax.dot_general(oh, evh_ref[...], dims,
                                 preferred_element_type=jnp.float32)
             + jax.lax.dot_general(oh, evl_ref[...], dims,
                                   preferred_element_type=jnp.float32))
        x = x + r.reshape(BB, S, H)

        mean = bcast_rows(jnp.mean(x, axis=-1))
        xc = x - mean
        var = bcast_rows(jnp.mean(xc * xc, axis=-1))
        o_ref[...] = (xc * lax.rsqrt(var + EPS) * bcast_lane(g_ref[...])
                      + bcast_lane(b_ref[...]))

    return pl.pallas_call(
        body,
        grid=(B // BB,),
        in_specs=[
            pl.BlockSpec((BB, S, H), lambda i: (i, 0, 0)),
            pl.BlockSpec((BB, S), lambda i: (i, 0)),
            pl.BlockSpec((BB, S), lambda i: (i, 0)),
            pl.BlockSpec((S, H), lambda i: (0, 0)),
            pl.BlockSpec((T, H), lambda i: (0, 0)),
            pl.BlockSpec((E, H), lambda i: (0, 0)),
            pl.BlockSpec((E, H), lambda i: (0, 0)),
            pl.BlockSpec((H,), lambda i: (0,)),
            pl.BlockSpec((H,), lambda i: (0,)),
        ],
        out_specs=pl.BlockSpec((BB, S, H), lambda i: (i, 0, 0)),
        out_shape=jax.ShapeDtypeStruct((B, S, H), jnp.float32),
    )(word_emb, tt_ids, ev_ids, pos_s, type_table, evt_hi, evt_lo,
      gamma, beta)


def kernel(input_ids, token_type_ids, event_type_ids, word_table, pos_table,
           type_table, event_table, gamma, beta):
    ids_flat = input_ids.reshape(BS).astype(jnp.int32)
    gathered = _sc_gather(word_table, ids_flat).reshape(B, S, H)
    evt_hi = event_table.astype(jnp.bfloat16)
    evt_lo = (event_table - evt_hi.astype(jnp.float32)).astype(jnp.bfloat16)
    return _tc_norm(gathered, token_type_ids, event_type_ids, pos_table[:S],
                    type_table, evt_hi, evt_lo, gamma, beta)
